# baseline (speedup 1.0000x reference)
import functools

import jax
import jax.numpy as jnp
from jax import lax
from jax.experimental import pallas as pl
from jax.experimental.pallas import tpu as pltpu
from jax.experimental.pallas import tpu_sc as plsc

_NC = 2
_NS = 16
_NW = _NC * _NS

_DIM = 32
_B = 16384
_S = 200
_TOK = 512
_BT = _TOK // 128


@jax.jit
def _embed_gather_t(xt_flat, table):
    mesh = plsc.VectorSubcoreMesh(core_axis_name="c", subcore_axis_name="s")

    @functools.partial(
        pl.kernel,
        mesh=mesh,
        out_type=jax.ShapeDtypeStruct(
            (_S, _DIM // 8, _B // 128, 8, 128), jnp.float32
        ),
        scratch_types=[
            pltpu.VMEM((2, _TOK), jnp.int32),
            pltpu.VMEM((2, _TOK, _DIM), jnp.float32),
            pltpu.VMEM((2, _BT, _DIM, 129), jnp.float32),
            pltpu.VMEM((2, _DIM // 8, _BT, 8, 128), jnp.float32),
            pltpu.SemaphoreType.DMA((2,)),
            pltpu.SemaphoreType.DMA((2,)),
            pltpu.SemaphoreType.DMA((2,)),
        ],
        compiler_params=pltpu.CompilerParams(
            use_tc_tiling_on_sc=False, needs_layout_passes=False
        ),
    )
    def k(
        xt_hbm, table_hbm, out_hbm, idx_v, rows_v, out_v, packed_v,
        isem, gsem, osem,
    ):
        wid = lax.axis_index("s") * _NC + lax.axis_index("c")
        b0 = wid * _TOK
        base_iota = lax.iota(jnp.int32, 16)
        row_ids = [base_iota + 16 * h for h in range(2)]
        btl_ids = [jnp.full((16,), b_, jnp.int32) for b_ in range(_BT)]

        def idx_copy(s, bi, sem):
            return pltpu.make_async_copy(
                xt_hbm.at[pl.ds(s * _B + b0, _TOK)], idx_v.at[bi], sem.at[bi]
            )

        def gather_copy(bi):
            return pltpu.make_async_copy(
                table_hbm.at[idx_v.at[bi]], rows_v.at[bi], gsem.at[bi]
            )

        def write_copy(s, bi):
            return pltpu.make_async_copy(
                packed_v.at[bi],
                out_hbm.at[s, :, pl.ds(_BT * wid, _BT)],
                osem.at[bi],
            )

        def start_write(s, bi):
            write_copy(s, bi).start()

        def wait_write(s, bi):
            write_copy(s, bi).wait()

        def repack(bi):
            def rbody(d, _):
                dt = lax.div(d, 8)
                di = lax.rem(d, 8)
                for btl in range(_BT):
                    for j in range(8):
                        v = out_v[bi, btl, d, pl.ds(16 * j, 16)]
                        packed_v[bi, dt, btl, di, pl.ds(16 * j, 16)] = v
                return 0

            lax.fori_loop(0, _DIM, rbody, 0)

        def transpose(bi):
            for btl in range(_BT):
                def tbody(tt, _, btl=btl):
                    t0 = btl * 128 + tt * 16
                    for k_ in range(16):
                        t = t0 + k_
                        col_v = jnp.broadcast_to(tt * 16 + k_, (16,))
                        for h in range(2):
                            v = rows_v[bi, t, pl.ds(16 * h, 16)]
                            plsc.store_scatter(
                                out_v.at[bi],
                                [btl_ids[btl], row_ids[h], col_v],
                                v,
                            )
                    return 0

                lax.fori_loop(0, 8, tbody, 0)

        idx_copy(0, 0, isem).start()
        idx_copy(0, 0, isem).wait()
        gather_copy(0).start()
        idx_copy(1, 1, isem).start()

        def body(i2, _):
            s0 = i2 * 2

            idx_copy(s0 + 1, 1, isem).wait()
            gather_copy(1).start()
            gather_copy(0).wait()

            @pl.when(i2 <= (_S // 2 - 2))
            def _():
                idx_copy(s0 + 2, 0, isem).start()

            @pl.when(i2 >= 1)
            def _():
                wait_write(s0 - 2, 0)

            transpose(0)
            repack(0)
            start_write(s0, 0)

            gather_copy(1).wait()

            @pl.when(i2 <= (_S // 2 - 2))
            def _():
                idx_copy(s0 + 2, 0, isem).wait()
                gather_copy(0).start()
                idx_copy(s0 + 3, 1, isem).start()

            @pl.when(i2 >= 1)
            def _():
                wait_write(s0 - 1, 1)

            transpose(1)
            repack(1)
            start_write(s0 + 1, 1)
            return 0

        lax.fori_loop(0, _S // 2, body, 0)
        wait_write(_S - 2, 0)
        wait_write(_S - 1, 1)

    return k(xt_flat, table)


def kernel(x, table):
    xt_flat = x.T.reshape(_S * _B).astype(jnp.int32)
    out5 = _embed_gather_t(xt_flat, table)
    return out5.transpose(2, 4, 0, 1, 3).reshape(_B, _S, _DIM)


# speedup vs baseline: 1.4710x; 1.4710x over previous
import functools

import jax
import jax.numpy as jnp
from jax import lax
from jax.experimental import pallas as pl
from jax.experimental.pallas import tpu as pltpu
from jax.experimental.pallas import tpu_sc as plsc

_NC = 2
_NS = 16
_NW = _NC * _NS

_DIM = 32
_B = 16384
_S = 200
_TOK = 512
_BT = _TOK // 128


@jax.jit
def _embed_gather_t(xt_flat, table):
    mesh = plsc.VectorSubcoreMesh(core_axis_name="c", subcore_axis_name="s")

    @functools.partial(
        pl.kernel,
        mesh=mesh,
        out_type=jax.ShapeDtypeStruct(
            (_S, _DIM // 8, _B // 128, 8, 128), jnp.float32
        ),
        scratch_types=[
            pltpu.VMEM((2, _TOK), jnp.int32),
            pltpu.VMEM((2, _TOK, _DIM), jnp.float32),
            pltpu.VMEM((2, _BT, _DIM // 8, 8, 129), jnp.float32),
            pltpu.SemaphoreType.DMA((2,)),
            pltpu.SemaphoreType.DMA((2,)),
            pltpu.SemaphoreType.DMA((2,)),
        ],
        compiler_params=pltpu.CompilerParams(
            use_tc_tiling_on_sc=False, needs_layout_passes=False
        ),
    )
    def k(xt_hbm, table_hbm, out_hbm, idx_v, rows_v, out_v, isem, gsem, osem):
        wid = lax.axis_index("s") * _NC + lax.axis_index("c")
        b0 = wid * _TOK
        base_iota = lax.iota(jnp.int32, 16)
        dt_ids = [(base_iota + 16 * h) // 8 for h in range(2)]
        di_ids = [lax.rem(base_iota + 16 * h, 8) for h in range(2)]

        def idx_copy(s, bi, sem):
            return pltpu.make_async_copy(
                xt_hbm.at[pl.ds(s * _B + b0, _TOK)], idx_v.at[bi], sem.at[bi]
            )

        def gather_copy(bi):
            return pltpu.make_async_copy(
                table_hbm.at[idx_v.at[bi]], rows_v.at[bi], gsem.at[bi]
            )

        def write_copies(s, bi):
            return [
                pltpu.make_async_copy(
                    out_v.at[bi, :, dt, :, pl.ds(0, 128)],
                    out_hbm.at[s, dt, pl.ds(_BT * wid, _BT)],
                    osem.at[bi],
                )
                for dt in range(_DIM // 8)
            ]

        def start_write(s, bi):
            for c in write_copies(s, bi):
                c.start()

        def wait_write(s, bi):
            for c in write_copies(s, bi):
                c.wait()

        def transpose(bi):
            def tbody(tt, _):
                t0 = tt * 32
                for k_ in range(32):
                    t = t0 + k_
                    btl = lax.div(t, 128)
                    col = lax.rem(t, 128)
                    btl_v = jnp.broadcast_to(btl, (16,))
                    col_v = jnp.broadcast_to(col, (16,))
                    for h in range(2):
                        v = rows_v[bi, t, pl.ds(16 * h, 16)]
                        plsc.store_scatter(
                            out_v.at[bi],
                            [btl_v, dt_ids[h], di_ids[h], col_v],
                            v,
                        )
                return 0

            lax.fori_loop(0, _TOK // 32, tbody, 0)

        idx_copy(0, 0, isem).start()
        idx_copy(0, 0, isem).wait()
        gather_copy(0).start()
        idx_copy(1, 1, isem).start()

        def body(i2, _):
            s0 = i2 * 2

            idx_copy(s0 + 1, 1, isem).wait()
            gather_copy(1).start()
            gather_copy(0).wait()

            @pl.when(i2 <= (_S // 2 - 2))
            def _():
                idx_copy(s0 + 2, 0, isem).start()

            @pl.when(i2 >= 1)
            def _():
                wait_write(s0 - 2, 0)

            transpose(0)
            start_write(s0, 0)

            gather_copy(1).wait()

            @pl.when(i2 <= (_S // 2 - 2))
            def _():
                idx_copy(s0 + 2, 0, isem).wait()
                gather_copy(0).start()
                idx_copy(s0 + 3, 1, isem).start()

            @pl.when(i2 >= 1)
            def _():
                wait_write(s0 - 1, 1)

            transpose(1)
            start_write(s0 + 1, 1)
            return 0

        lax.fori_loop(0, _S // 2, body, 0)
        wait_write(_S - 2, 0)
        wait_write(_S - 1, 1)

    return k(xt_flat, table)


def kernel(x, table):
    xt_flat = x.T.reshape(_S * _B).astype(jnp.int32)
    out5 = _embed_gather_t(xt_flat, table)
    return out5.transpose(2, 4, 0, 1, 3).reshape(_B, _S, _DIM)


# speedup vs baseline: 1.5126x; 1.0283x over previous
import functools

import jax
import jax.numpy as jnp
from jax import lax
from jax.experimental import pallas as pl
from jax.experimental.pallas import tpu as pltpu
from jax.experimental.pallas import tpu_sc as plsc

_NC = 2
_NS = 16
_NW = _NC * _NS

_DIM = 32
_B = 16384
_S = 200
_TOK = 512
_BT = _TOK // 128


@jax.jit
def _embed_gather_t(xt_flat, table):
    mesh = plsc.VectorSubcoreMesh(core_axis_name="c", subcore_axis_name="s")

    @functools.partial(
        pl.kernel,
        mesh=mesh,
        out_type=jax.ShapeDtypeStruct(
            (_S, _DIM // 8, _B // 128, 8, 128), jnp.float32
        ),
        scratch_types=[
            pltpu.VMEM((2, _TOK), jnp.int32),
            pltpu.VMEM((2, _TOK, _DIM), jnp.float32),
            pltpu.VMEM((2, _BT, _DIM // 8, 8, 129), jnp.float32),
            pltpu.SemaphoreType.DMA((2,)),
            pltpu.SemaphoreType.DMA((2,)),
            pltpu.SemaphoreType.DMA((2,)),
        ],
        compiler_params=pltpu.CompilerParams(
            use_tc_tiling_on_sc=False, needs_layout_passes=False
        ),
    )
    def k(xt_hbm, table_hbm, out_hbm, idx_v, rows_v, out_v, isem, gsem, osem):
        wid = lax.axis_index("s") * _NC + lax.axis_index("c")
        b0 = wid * _TOK
        base_iota = lax.iota(jnp.int32, 16)
        dt_ids = [(base_iota + 16 * h) // 8 for h in range(2)]
        di_ids = [lax.rem(base_iota + 16 * h, 8) for h in range(2)]

        def idx_copy(s, bi, sem):
            return pltpu.make_async_copy(
                xt_hbm.at[pl.ds(s * _B + b0, _TOK)], idx_v.at[bi], sem.at[bi]
            )

        def gather_copy(bi):
            return pltpu.make_async_copy(
                table_hbm.at[idx_v.at[bi]], rows_v.at[bi], gsem.at[bi]
            )

        def write_copies(s, bi):
            return [
                pltpu.make_async_copy(
                    out_v.at[bi, :, dt, :, pl.ds(0, 128)],
                    out_hbm.at[s, dt, pl.ds(_BT * wid, _BT)],
                    osem.at[bi],
                )
                for dt in range(_DIM // 8)
            ]

        def start_write(s, bi):
            for c in write_copies(s, bi):
                c.start()

        def wait_write(s, bi):
            for c in write_copies(s, bi):
                c.wait()

        def transpose(bi):
            def tbody(tt, _):
                t0 = tt * 16
                for k_ in range(16):
                    t = t0 + k_
                    btl = lax.div(t, 128)
                    col = lax.rem(t, 128)
                    btl_v = jnp.broadcast_to(btl, (16,))
                    col_v = jnp.broadcast_to(col, (16,))
                    for h in range(2):
                        v = rows_v[bi, t, pl.ds(16 * h, 16)]
                        plsc.store_scatter(
                            out_v.at[bi],
                            [btl_v, dt_ids[h], di_ids[h], col_v],
                            v,
                        )
                return 0

            lax.fori_loop(0, _TOK // 16, tbody, 0)

        idx_copy(0, 0, isem).start()
        idx_copy(0, 0, isem).wait()
        gather_copy(0).start()
        idx_copy(1, 1, isem).start()

        def body(i2, _):
            s0 = i2 * 2

            idx_copy(s0 + 1, 1, isem).wait()
            gather_copy(1).start()
            gather_copy(0).wait()

            @pl.when(i2 <= (_S // 2 - 2))
            def _():
                idx_copy(s0 + 2, 0, isem).start()

            @pl.when(i2 >= 1)
            def _():
                wait_write(s0 - 2, 0)

            transpose(0)
            start_write(s0, 0)

            gather_copy(1).wait()

            @pl.when(i2 <= (_S // 2 - 2))
            def _():
                idx_copy(s0 + 2, 0, isem).wait()
                gather_copy(0).start()
                idx_copy(s0 + 3, 1, isem).start()

            @pl.when(i2 >= 1)
            def _():
                wait_write(s0 - 1, 1)

            transpose(1)
            start_write(s0 + 1, 1)
            return 0

        lax.fori_loop(0, _S // 2, body, 0)
        wait_write(_S - 2, 0)
        wait_write(_S - 1, 1)

    return k(xt_flat, table)


def kernel(x, table):
    xt_flat = x.T.reshape(_S * _B).astype(jnp.int32)
    out5 = _embed_gather_t(xt_flat, table)
    return out5.transpose(2, 4, 0, 1, 3).reshape(_B, _S, _DIM)


# speedup vs baseline: 2.4347x; 1.6096x over previous
import functools

import jax
import jax.numpy as jnp
from jax import lax
from jax.experimental import pallas as pl
from jax.experimental.pallas import tpu as pltpu
from jax.experimental.pallas import tpu_sc as plsc

_NC = 2
_NS = 16
_NW = _NC * _NS

_DIM = 32
_B = 16384
_S = 200
_TOK = 512
_BT = _TOK // 128


@jax.jit
def _embed_gather_t(xt_flat, table):
    mesh = plsc.VectorSubcoreMesh(core_axis_name="c", subcore_axis_name="s")

    @functools.partial(
        pl.kernel,
        mesh=mesh,
        out_type=jax.ShapeDtypeStruct(
            (_S, _DIM // 8, _B // 128, 8, 128), jnp.float32
        ),
        scratch_types=[
            pltpu.VMEM((2, _TOK), jnp.int32),
            pltpu.VMEM((2, _TOK, _DIM), jnp.float32),
            pltpu.VMEM((2, _BT, _DIM // 8, 8, 129), jnp.float32),
            pltpu.SemaphoreType.DMA((2,)),
            pltpu.SemaphoreType.DMA((2,)),
            pltpu.SemaphoreType.DMA((2,)),
        ],
        compiler_params=pltpu.CompilerParams(
            use_tc_tiling_on_sc=False, needs_layout_passes=False
        ),
    )
    def k(xt_hbm, table_hbm, out_hbm, idx_v, rows_v, out_v, isem, gsem, osem):
        wid = lax.axis_index("s") * _NC + lax.axis_index("c")
        b0 = wid * _TOK
        base_iota = lax.iota(jnp.int32, 16)
        dt_ids = [(base_iota + 16 * h) // 8 for h in range(2)]
        di_ids = [lax.rem(base_iota + 16 * h, 8) for h in range(2)]

        def idx_copy(s, bi, sem):
            return pltpu.make_async_copy(
                xt_hbm.at[pl.ds(s * _B + b0, _TOK)], idx_v.at[bi], sem.at[bi]
            )

        def gather_copy(bi):
            return pltpu.make_async_copy(
                table_hbm.at[idx_v.at[bi]], rows_v.at[bi], gsem.at[bi]
            )

        def write_copies(s, bi):
            return [
                pltpu.make_async_copy(
                    out_v.at[bi, :, dt, :, pl.ds(0, 128)],
                    out_hbm.at[s, dt, pl.ds(_BT * wid, _BT)],
                    osem.at[bi],
                )
                for dt in range(_DIM // 8)
            ]

        def start_write(s, bi):
            for c in write_copies(s, bi):
                c.start()

        def wait_write(s, bi):
            for c in write_copies(s, bi):
                c.wait()

        def transpose(bi):
            @functools.partial(plsc.parallel_loop, 0, _TOK // 16)
            def tbody(tt):
                t0 = tt * 16
                for k_ in range(16):
                    t = t0 + k_
                    btl = lax.div(t, 128)
                    col = lax.rem(t, 128)
                    btl_v = jnp.broadcast_to(btl, (16,))
                    col_v = jnp.broadcast_to(col, (16,))
                    for h in range(2):
                        v = rows_v[bi, t, pl.ds(16 * h, 16)]
                        plsc.store_scatter(
                            out_v.at[bi],
                            [btl_v, dt_ids[h], di_ids[h], col_v],
                            v,
                        )

        idx_copy(0, 0, isem).start()
        idx_copy(0, 0, isem).wait()
        gather_copy(0).start()
        idx_copy(1, 1, isem).start()

        def body(i2, _):
            s0 = i2 * 2

            idx_copy(s0 + 1, 1, isem).wait()
            gather_copy(1).start()
            gather_copy(0).wait()

            @pl.when(i2 <= (_S // 2 - 2))
            def _():
                idx_copy(s0 + 2, 0, isem).start()

            @pl.when(i2 >= 1)
            def _():
                wait_write(s0 - 2, 0)

            transpose(0)
            start_write(s0, 0)

            gather_copy(1).wait()

            @pl.when(i2 <= (_S // 2 - 2))
            def _():
                idx_copy(s0 + 2, 0, isem).wait()
                gather_copy(0).start()
                idx_copy(s0 + 3, 1, isem).start()

            @pl.when(i2 >= 1)
            def _():
                wait_write(s0 - 1, 1)

            transpose(1)
            start_write(s0 + 1, 1)
            return 0

        lax.fori_loop(0, _S // 2, body, 0)
        wait_write(_S - 2, 0)
        wait_write(_S - 1, 1)

    return k(xt_flat, table)


def kernel(x, table):
    xt_flat = x.T.reshape(_S * _B).astype(jnp.int32)
    out5 = _embed_gather_t(xt_flat, table)
    return out5.transpose(2, 4, 0, 1, 3).reshape(_B, _S, _DIM)


# speedup vs baseline: 2.4380x; 1.0014x over previous
import functools

import jax
import jax.numpy as jnp
from jax import lax
from jax.experimental import pallas as pl
from jax.experimental.pallas import tpu as pltpu
from jax.experimental.pallas import tpu_sc as plsc

_NC = 2
_NS = 16
_NW = _NC * _NS

_DIM = 32
_B = 16384
_S = 200
_TOK = 512
_BT = _TOK // 128


@jax.jit
def _embed_gather_t(xt_flat, table):
    mesh = plsc.VectorSubcoreMesh(core_axis_name="c", subcore_axis_name="s")

    @functools.partial(
        pl.kernel,
        mesh=mesh,
        out_type=jax.ShapeDtypeStruct(
            (_S, _DIM // 8, _B // 128, 8, 128), jnp.float32
        ),
        scratch_types=[
            pltpu.VMEM((2, _TOK), jnp.int32),
            pltpu.VMEM((2, _TOK, _DIM), jnp.float32),
            pltpu.VMEM((2, _BT, _DIM // 8, 8, 129), jnp.float32),
            pltpu.SemaphoreType.DMA((2,)),
            pltpu.SemaphoreType.DMA((2,)),
            pltpu.SemaphoreType.DMA((2,)),
        ],
        compiler_params=pltpu.CompilerParams(
            use_tc_tiling_on_sc=False, needs_layout_passes=False
        ),
    )
    def k(xt_hbm, table_hbm, out_hbm, idx_v, rows_v, out_v, isem, gsem, osem):
        wid = lax.axis_index("s") * _NC + lax.axis_index("c")
        b0 = wid * _TOK
        base_iota = lax.iota(jnp.int32, 16)
        dt_ids = [(base_iota + 16 * h) // 8 for h in range(2)]
        di_ids = [lax.rem(base_iota + 16 * h, 8) for h in range(2)]

        def idx_copy(s, bi, sem):
            return pltpu.make_async_copy(
                xt_hbm.at[pl.ds(s * _B + b0, _TOK)], idx_v.at[bi], sem.at[bi]
            )

        def gather_copy(bi):
            return pltpu.make_async_copy(
                table_hbm.at[idx_v.at[bi]], rows_v.at[bi], gsem.at[bi]
            )

        def write_copies(s, bi):
            return [
                pltpu.make_async_copy(
                    out_v.at[bi, :, dt, :, pl.ds(0, 128)],
                    out_hbm.at[s, dt, pl.ds(_BT * wid, _BT)],
                    osem.at[bi],
                )
                for dt in range(_DIM // 8)
            ]

        def start_write(s, bi):
            for c in write_copies(s, bi):
                c.start()

        def wait_write(s, bi):
            for c in write_copies(s, bi):
                c.wait()

        def transpose(bi):
            @functools.partial(plsc.parallel_loop, 0, _TOK // 16)
            def tbody(tt):
                t0 = tt * 16
                for k_ in range(16):
                    t = t0 + k_
                    btl = lax.div(t, 128)
                    col = lax.rem(t, 128)
                    btl_v = jnp.broadcast_to(btl, (16,))
                    col_v = jnp.broadcast_to(col, (16,))
                    for h in range(2):
                        v = rows_v[bi, t, pl.ds(16 * h, 16)]
                        plsc.store_scatter(
                            out_v.at[bi],
                            [btl_v, dt_ids[h], di_ids[h], col_v],
                            v,
                        )

        idx_copy(0, 0, isem).start()
        idx_copy(0, 0, isem).wait()
        gather_copy(0).start()
        idx_copy(1, 1, isem).start()

        def body(i2, _):
            s0 = i2 * 2

            idx_copy(s0 + 1, 1, isem).wait()
            gather_copy(1).start()
            gather_copy(0).wait()

            @pl.when(i2 <= (_S // 2 - 2))
            def _():
                idx_copy(s0 + 2, 0, isem).start()

            @pl.when(i2 >= 1)
            def _():
                wait_write(s0 - 2, 0)

            transpose(0)
            start_write(s0, 0)

            gather_copy(1).wait()

            @pl.when(i2 <= (_S // 2 - 2))
            def _():
                idx_copy(s0 + 2, 0, isem).wait()
                gather_copy(0).start()
                idx_copy(s0 + 3, 1, isem).start()

            @pl.when(i2 >= 1)
            def _():
                wait_write(s0 - 1, 1)

            transpose(1)
            start_write(s0 + 1, 1)
            return 0

        lax.fori_loop(0, _S // 2, body, 0)
        wait_write(_S - 2, 0)
        wait_write(_S - 1, 1)

    return k(xt_flat, table)


def kernel(x, table):
    xt_flat = x.T.reshape(_S * _B).astype(jnp.int32)
    t4 = lax.optimization_barrier(table.reshape(-1, 128))
    out5 = _embed_gather_t(xt_flat, t4.reshape(-1, _DIM))
    return out5.transpose(2, 4, 0, 1, 3).reshape(_B, _S, _DIM)
